# baseline (speedup 1.0000x reference)
import functools

import jax
import jax.numpy as jnp
from jax import lax
from jax.experimental import pallas as pl
from jax.experimental.pallas import tpu as pltpu
from jax.experimental.pallas import tpu_sc as plsc

_NUM_CORES = 2
_NUM_SUBCORES = 16
_NW = _NUM_CORES * _NUM_SUBCORES


def _sc_gather(table, idx, chunk):
  n, = idx.shape
  _, d = table.shape
  per_w = n // _NW
  n_chunks = per_w // chunk
  assert per_w % chunk == 0 and chunk % 8 == 0

  mesh = plsc.VectorSubcoreMesh(core_axis_name="c", subcore_axis_name="s")

  @functools.partial(
      pl.kernel,
      out_type=jax.ShapeDtypeStruct((n, d), jnp.float32),
      mesh=mesh,
      scratch_types=[
          pltpu.VMEM((chunk,), jnp.int32),
          pltpu.VMEM((chunk, d), jnp.float32),
          pltpu.SemaphoreType.DMA,
      ],
      compiler_params=pltpu.CompilerParams(use_tc_tiling_on_sc=False),
  )
  def gather_kernel(table_hbm, idx_hbm, out_hbm, idx_v, rows_v, sem):
    wid = lax.axis_index("s") * _NUM_CORES + lax.axis_index("c")
    base = wid * per_w

    def body(g, carry):
      off = base + g * chunk
      pltpu.sync_copy(idx_hbm.at[pl.ds(off, chunk)], idx_v)
      pltpu.async_copy(table_hbm.at[idx_v], rows_v, sem).wait()
      pltpu.sync_copy(rows_v, out_hbm.at[pl.ds(off, chunk)])
      return carry

    lax.fori_loop(0, n_chunks, body, 0)

  return gather_kernel(table, idx)


def _tc_mlp(x_num, emb3, w1n, w1c, b1, w2, b2, w3, b3, wab, bab, bm):
  b, nd = x_num.shape
  grid = (b // bm,)

  def body(xn_ref, emb_ref, w1n_ref, w1c_ref, b1_ref, w2_ref, b2_ref,
           w3_ref, b3_ref, wab_ref, bab_ref, out_ref):
    h = jnp.dot(xn_ref[...], w1n_ref[...], preferred_element_type=jnp.float32)
    for g in range(4):
      h = h + jnp.dot(emb_ref[g], w1c_ref[g],
                      preferred_element_type=jnp.float32)
    h = jnp.maximum(h + b1_ref[...], 0.0)
    h = jnp.maximum(
        jnp.dot(h, w2_ref[...], preferred_element_type=jnp.float32)
        + b2_ref[...], 0.0)
    h = jnp.maximum(
        jnp.dot(h, w3_ref[...], preferred_element_type=jnp.float32)
        + b3_ref[...], 0.0)
    out_ref[...] = (
        jnp.dot(h, wab_ref[...], preferred_element_type=jnp.float32)
        + bab_ref[...])

  full2 = lambda shape: pl.BlockSpec(shape, lambda i: (0, 0))
  full3 = lambda shape: pl.BlockSpec(shape, lambda i: (0, 0, 0))
  return pl.pallas_call(
      body,
      grid=grid,
      in_specs=[
          pl.BlockSpec((bm, nd), lambda i: (i, 0)),
          pl.BlockSpec((4, bm, 128), lambda i: (0, i, 0)),
          full2(w1n.shape),
          full3(w1c.shape),
          full2(b1.shape),
          full2(w2.shape),
          full2(b2.shape),
          full2(w3.shape),
          full2(b3.shape),
          full2(wab.shape),
          full2(bab.shape),
      ],
      out_specs=pl.BlockSpec((bm, 2), lambda i: (i, 0)),
      out_shape=jax.ShapeDtypeStruct((b, 2), jnp.float32),
  )(x_num, emb3, w1n, w1c, b1, w2, b2, w3, b3, wab, bab)


def kernel(x_num, x_cat, E, W1, b1, W2, b2, W3, b3, WA, bA, WB, bB):
  f, v, d = E.shape
  b = x_cat.shape[0]
  nd = x_num.shape[1]
  fp = 32

  table = E.reshape(f * v, d)
  idx2 = x_cat + (jnp.arange(f, dtype=jnp.int32) * v)[None, :]
  idx2 = jnp.pad(idx2, ((0, 0), (0, fp - f)))
  idx3 = idx2.reshape(b, 4, 8).transpose(1, 0, 2).reshape(-1)

  emb = _sc_gather(table, idx3, chunk=2048)
  emb3 = emb.reshape(4, b, 8 * d)

  w1e = jnp.pad(W1[nd:], ((0, fp * d - f * d), (0, 0)))
  w1c = w1e.reshape(4, 8 * d, W1.shape[1])
  wab = jnp.concatenate([WA, WB], axis=1)
  bab = jnp.concatenate([bA, bB])[None, :]
  out = _tc_mlp(x_num, emb3, W1[:nd], w1c, b1[None, :], W2, b2[None, :],
                W3, b3[None, :], wab, bab, bm=2048)
  return out[:, 0], out[:, 1]


# speedup vs baseline: 1.1680x; 1.1680x over previous
import functools

import jax
import jax.numpy as jnp
from jax import lax
from jax.experimental import pallas as pl
from jax.experimental.pallas import tpu as pltpu
from jax.experimental.pallas import tpu_sc as plsc

_NUM_CORES = 2
_NUM_SUBCORES = 16
_NW = _NUM_CORES * _NUM_SUBCORES
_L = 16
_CHUNK = 512


def _sc_gather(table8, idx):
  n, = idx.shape
  per_w = n // _NW
  n_chunks = per_w // _CHUNK
  assert per_w % _CHUNK == 0

  mesh = plsc.VectorSubcoreMesh(core_axis_name="c", subcore_axis_name="s")

  @functools.partial(
      pl.kernel,
      out_type=jax.ShapeDtypeStruct((n * 16 // 128, 128), jnp.float32),
      mesh=mesh,
      scratch_types=[
          pltpu.VMEM((per_w,), jnp.int32),
          pltpu.VMEM((per_w,), jnp.int32),
          pltpu.VMEM((per_w,), jnp.int32),
          pltpu.VMEM((_CHUNK, 128), jnp.float32),
          pltpu.VMEM((_CHUNK // 8, 128), jnp.float32),
          pltpu.SemaphoreType.DMA,
      ],
      compiler_params=pltpu.CompilerParams(use_tc_tiling_on_sc=True,
                                           needs_layout_passes=False),
  )
  def gather_kernel(tab_hbm, idx_hbm, out_hbm, idx_v, q_v, r_v, buf_v, out_v,
                    sem):
    wid = lax.axis_index("s") * _NUM_CORES + lax.axis_index("c")
    base = pl.multiple_of(wid * per_w, per_w)

    pltpu.sync_copy(idx_hbm.at[pl.ds(base, per_w)], idx_v)

    def split_body(i, carry):
      ix = idx_v[pl.ds(i * _L, _L)]
      q_v[pl.ds(i * _L, _L)] = lax.shift_right_logical(ix, 3)
      r_v[pl.ds(i * _L, _L)] = lax.shift_left(jnp.bitwise_and(ix, 7), 4)
      return carry

    lax.fori_loop(0, per_w // _L, split_body, 0)

    lanes = lax.iota(jnp.int32, _L)

    def chunk_body(c, carry):
      off = c * _CHUNK
      pltpu.async_copy(tab_hbm.at[q_v.at[pl.ds(off, _CHUNK)]], buf_v,
                       sem).wait()

      def group_body(g, carry2):
        i0 = g * _L
        ivec = lanes + i0
        rvec = r_v[pl.ds(off + i0, _L)]
        orow = lax.shift_right_logical(ivec, 3)
        ocol = lax.shift_left(jnp.bitwise_and(ivec, 7), 4)
        for w in range(16):
          vals = plsc.load_gather(buf_v, [ivec, rvec + w])
          plsc.store_scatter(out_v, [orow, ocol + w], vals)
        return carry2

      lax.fori_loop(0, _CHUNK // _L, group_body, 0)
      pltpu.sync_copy(
          out_v,
          out_hbm.at[pl.ds(pl.multiple_of((base + off) // 8, _CHUNK // 8),
                           _CHUNK // 8)])
      return carry

    lax.fori_loop(0, n_chunks, chunk_body, 0)

  return gather_kernel(table8, idx)


def _tc_mlp(x_num, emb3, w1n, w1c, b1, w2, b2, w3, b3, wab, bab, bm):
  b, nd = x_num.shape
  grid = (b // bm,)

  def body(xn_ref, emb_ref, w1n_ref, w1c_ref, b1_ref, w2_ref, b2_ref,
           w3_ref, b3_ref, wab_ref, bab_ref, out_ref):
    h = jnp.dot(xn_ref[...], w1n_ref[...], preferred_element_type=jnp.float32)
    for g in range(4):
      h = h + jnp.dot(emb_ref[g], w1c_ref[g],
                      preferred_element_type=jnp.float32)
    h = jnp.maximum(h + b1_ref[...], 0.0)
    h = jnp.maximum(
        jnp.dot(h, w2_ref[...], preferred_element_type=jnp.float32)
        + b2_ref[...], 0.0)
    h = jnp.maximum(
        jnp.dot(h, w3_ref[...], preferred_element_type=jnp.float32)
        + b3_ref[...], 0.0)
    out_ref[...] = (
        jnp.dot(h, wab_ref[...], preferred_element_type=jnp.float32)
        + bab_ref[...])

  full2 = lambda shape: pl.BlockSpec(shape, lambda i: (0, 0))
  full3 = lambda shape: pl.BlockSpec(shape, lambda i: (0, 0, 0))
  return pl.pallas_call(
      body,
      grid=grid,
      in_specs=[
          pl.BlockSpec((bm, nd), lambda i: (i, 0)),
          pl.BlockSpec((4, bm, 128), lambda i: (0, i, 0)),
          full2(w1n.shape),
          full3(w1c.shape),
          full2(b1.shape),
          full2(w2.shape),
          full2(b2.shape),
          full2(w3.shape),
          full2(b3.shape),
          full2(wab.shape),
          full2(bab.shape),
      ],
      out_specs=pl.BlockSpec((bm, 2), lambda i: (i, 0)),
      out_shape=jax.ShapeDtypeStruct((b, 2), jnp.float32),
  )(x_num, emb3, w1n, w1c, b1, w2, b2, w3, b3, wab, bab)


def kernel(x_num, x_cat, E, W1, b1, W2, b2, W3, b3, WA, bA, WB, bB):
  f, v, d = E.shape
  b = x_cat.shape[0]
  nd = x_num.shape[1]
  fp = 32

  table8 = E.reshape(f * v // 8, 8 * d)
  idx2 = x_cat + (jnp.arange(f, dtype=jnp.int32) * v)[None, :]
  idx2 = jnp.concatenate([idx2, idx2[:, :fp - f]], axis=1)
  idx3 = idx2.reshape(b, 4, 8).transpose(1, 0, 2).reshape(-1)

  emb = _sc_gather(table8, idx3)
  emb3 = emb.reshape(4, b, 8 * d)

  w1e = jnp.pad(W1[nd:], ((0, fp * d - f * d), (0, 0)))
  w1c = w1e.reshape(4, 8 * d, W1.shape[1])
  wab = jnp.concatenate([WA, WB], axis=1)
  bab = jnp.concatenate([bA, bB])[None, :]
  out = _tc_mlp(x_num, emb3, W1[:nd], w1c, b1[None, :], W2, b2[None, :],
                W3, b3[None, :], wab, bab, bm=2048)
  return out[:, 0], out[:, 1]
